# Initial kernel scaffold; baseline (speedup 1.0000x reference)
import jax
import jax.numpy as jnp
from jax.experimental import pallas as pl


def kernel(input_ids, emb, W_ih, W_hh, b_ih, b_hh, W_he, b_he, out_bias, w_read, b_read, w_write, b_write, w_decay, b_decay, mem_scale):
    raise NotImplementedError("write your pallas kernel here")


# speedup vs baseline: 27.0973x; 27.0973x over previous
import functools

import jax
import jax.numpy as jnp
from jax import lax
from jax.experimental import pallas as pl
from jax.experimental.pallas import tpu as pltpu

V = 100000
D = 64
H = 128
B = 8
S = 32
N = B * S
VBLK = 2048


def _coeffs_body(ids_smem, emb_hbm, ids_vmem, ids_col, W_ih, W_hh, b_ih, b_hh,
                 W_he, b_he, wrd, bias3, mscale,
                 proj_ref, a3_ref,
                 embs_ref, xproj_ref, states_ref, sem):
    def _fire(i, _):
        t = i // B
        b = i % B
        tok = ids_smem[b, t]
        pltpu.make_async_copy(
            emb_hbm.at[pl.ds(tok, 1), :], embs_ref.at[pl.ds(i, 1), :], sem
        ).start()
        return 0

    lax.fori_loop(0, N, _fire, 0)

    def _drain(i, _):
        t = i // B
        b = i % B
        tok = ids_smem[b, t]
        pltpu.make_async_copy(
            emb_hbm.at[pl.ds(tok, 1), :], embs_ref.at[pl.ds(i, 1), :], sem
        ).wait()
        return 0

    lax.fori_loop(0, N, _drain, 0)

    embs = embs_ref[...]
    xproj_ref[...] = lax.dot_general(
        embs, W_ih[...], (((1,), (1,)), ((), ())),
        preferred_element_type=jnp.float32) + b_ih[...]

    Whh = W_hh[...]
    bhh = b_hh[...]

    def _step(t, h):
        xg = xproj_ref[pl.ds(t * B, B), :]
        hg = lax.dot_general(h, Whh, (((1,), (1,)), ((), ())),
                             preferred_element_type=jnp.float32) + bhh
        xr, xz, xn = xg[:, :H], xg[:, H:2 * H], xg[:, 2 * H:]
        hr, hz, hn = hg[:, :H], hg[:, H:2 * H], hg[:, 2 * H:]
        r = jax.nn.sigmoid(xr + hr)
        z = jax.nn.sigmoid(xz + hz)
        nn_ = jnp.tanh(xn + r * hn)
        h_new = (1.0 - z) * nn_ + z * h
        states_ref[:, pl.ds(t, 1), :] = h_new.reshape(B, 1, H)
        return h_new

    lax.fori_loop(0, S, _step, jnp.zeros((B, H), jnp.float32))

    br = bias3[0, 0]
    bw = bias3[0, 1]
    bd = bias3[0, 2]
    ms = mscale[0, 0]
    Whe = W_he[...]
    bhe = b_he[...]
    wrd_v = wrd[...]

    iota_t = lax.broadcasted_iota(jnp.int32, (S, S), 0)
    iota_j = lax.broadcasted_iota(jnp.int32, (S, S), 1)
    mask_lt = iota_j < iota_t
    lt_le_col = jnp.where(iota_j < iota_t, 1.0, 0.0)
    le_row = jnp.where(iota_t <= iota_j, 1.0, 0.0)

    for b in range(B):
        st_b = states_ref[b, :, :]
        proj_ref[b * S:(b + 1) * S, :] = lax.dot_general(
            st_b, Whe, (((1,), (1,)), ((), ())),
            preferred_element_type=jnp.float32) + bhe
        g_col = lax.dot_general(st_b, wrd_v, (((1,), (1,)), ((), ())),
                                preferred_element_type=jnp.float32)
        g_row = lax.dot_general(wrd_v, st_b, (((1,), (1,)), ((), ())),
                                preferred_element_type=jnp.float32)
        read_col = jax.nn.sigmoid(g_col[:, 0:1] + br) * ms
        write_row = jax.nn.sigmoid(g_row[1:2, :] + bw)
        dec_col = jax.nn.sigmoid(g_col[:, 2:3] + bd)
        dec_row = jax.nn.sigmoid(g_row[2:3, :] + bd)
        ld_col = jnp.log(jnp.maximum(dec_col, 1e-30))
        ld_row = jnp.log(jnp.maximum(dec_row, 1e-30))
        cum_row = lax.dot_general(ld_row, le_row, (((1,), (0,)), ((), ())),
                                  preferred_element_type=jnp.float32)
        cum_tm1_col = lax.dot_general(lt_le_col, ld_col,
                                      (((1,), (0,)), ((), ())),
                                      preferred_element_type=jnp.float32)
        arg = jnp.where(mask_lt, cum_tm1_col - cum_row, -1e30)
        a3_ref[b, :, :] = jnp.exp(arg) * write_row * read_col

    del ids_vmem, ids_col


def _logits_body(proj, emb_blk, bias_blk, a3, ids_col, out_ref):
    base = lax.dot_general(
        proj[...], emb_blk[...], (((1,), (1,)), ((), ())),
        preferred_element_type=jnp.float32) + bias_blk[...]
    v0 = pl.program_id(0) * VBLK
    col = lax.broadcasted_iota(jnp.int32, (S, VBLK), 1) + v0
    for b in range(B):
        idb = ids_col[b, :, :]
        oh = jnp.where(col == idb, 1.0, 0.0)
        corr = lax.dot_general(a3[b, :, :], oh, (((1,), (0,)), ((), ())),
                               preferred_element_type=jnp.float32)
        out_ref[b * S:(b + 1) * S, :] = base[b * S:(b + 1) * S, :] + corr


def kernel(input_ids, emb, W_ih, W_hh, b_ih, b_hh, W_he, b_he, out_bias,
           w_read, b_read, w_write, b_write, w_decay, b_decay, mem_scale):
    ids = input_ids.astype(jnp.int32)
    ids_col = ids.reshape(B, S, 1)
    wrd = jnp.concatenate([w_read, w_write, w_decay], axis=0)
    bias3 = jnp.stack([b_read[0], b_write[0], b_decay[0]]).reshape(1, 3)
    mscale = mem_scale.reshape(1, 1)

    proj, a3 = pl.pallas_call(
        _coeffs_body,
        in_specs=[
            pl.BlockSpec(memory_space=pltpu.SMEM),
            pl.BlockSpec(memory_space=pltpu.HBM),
            pl.BlockSpec(memory_space=pltpu.VMEM),
            pl.BlockSpec(memory_space=pltpu.VMEM),
            pl.BlockSpec(memory_space=pltpu.VMEM),
            pl.BlockSpec(memory_space=pltpu.VMEM),
            pl.BlockSpec(memory_space=pltpu.VMEM),
            pl.BlockSpec(memory_space=pltpu.VMEM),
            pl.BlockSpec(memory_space=pltpu.VMEM),
            pl.BlockSpec(memory_space=pltpu.VMEM),
            pl.BlockSpec(memory_space=pltpu.VMEM),
            pl.BlockSpec(memory_space=pltpu.SMEM),
            pl.BlockSpec(memory_space=pltpu.SMEM),
        ],
        out_shape=[
            jax.ShapeDtypeStruct((N, D), jnp.float32),
            jax.ShapeDtypeStruct((B, S, S), jnp.float32),
        ],
        scratch_shapes=[
            pltpu.VMEM((N, D), jnp.float32),
            pltpu.VMEM((N, 3 * H), jnp.float32),
            pltpu.VMEM((B, S, H), jnp.float32),
            pltpu.SemaphoreType.DMA,
        ],
    )(ids, emb, ids, ids_col, W_ih, W_hh, b_ih.reshape(1, 3 * H),
      b_hh.reshape(1, 3 * H), W_he, b_he.reshape(1, D), wrd, bias3, mscale)

    nblk = (V + VBLK - 1) // VBLK
    out2d = pl.pallas_call(
        _logits_body,
        grid=(nblk,),
        in_specs=[
            pl.BlockSpec((N, D), lambda i: (0, 0)),
            pl.BlockSpec((VBLK, D), lambda i: (i, 0)),
            pl.BlockSpec((1, VBLK), lambda i: (0, i)),
            pl.BlockSpec((B, S, S), lambda i: (0, 0, 0)),
            pl.BlockSpec((B, S, 1), lambda i: (0, 0, 0)),
        ],
        out_specs=pl.BlockSpec((N, VBLK), lambda i: (0, i)),
        out_shape=jax.ShapeDtypeStruct((N, V), jnp.float32),
    )(proj, emb, out_bias.reshape(1, V), a3, ids_col)

    return out2d.reshape(B, S, V)


# speedup vs baseline: 31.8995x; 1.1772x over previous
import functools

import jax
import jax.numpy as jnp
from jax import lax
from jax.experimental import pallas as pl
from jax.experimental.pallas import tpu as pltpu

V = 100000
D = 64
H = 128
B = 8
S = 32
N = B * S
VBLK = 8192


def _coeffs_body(ids_smem, emb_hbm, ids_vmem, ids_col, W_ih, W_hh, b_ih, b_hh,
                 W_he, b_he, wrd, bias3, mscale,
                 proj_ref, a3_ref,
                 embs_ref, xproj_ref, states_ref, sem):
    def _fire(i, _):
        t = i // B
        b = i % B
        tok = ids_smem[b, t]
        pltpu.make_async_copy(
            emb_hbm.at[pl.ds(tok, 1), :], embs_ref.at[pl.ds(i, 1), :], sem
        ).start()
        return 0

    lax.fori_loop(0, N, _fire, 0)

    def _drain(i, _):
        t = i // B
        b = i % B
        tok = ids_smem[b, t]
        pltpu.make_async_copy(
            emb_hbm.at[pl.ds(tok, 1), :], embs_ref.at[pl.ds(i, 1), :], sem
        ).wait()
        return 0

    lax.fori_loop(0, N, _drain, 0)

    embs = embs_ref[...]
    xproj_ref[...] = lax.dot_general(
        embs, W_ih[...], (((1,), (1,)), ((), ())),
        preferred_element_type=jnp.float32) + b_ih[...]

    Whh = W_hh[...]
    bhh = b_hh[...]

    def _step(t, h):
        xg = xproj_ref[pl.ds(t * B, B), :]
        hg = lax.dot_general(h, Whh, (((1,), (1,)), ((), ())),
                             preferred_element_type=jnp.float32) + bhh
        xr, xz, xn = xg[:, :H], xg[:, H:2 * H], xg[:, 2 * H:]
        hr, hz, hn = hg[:, :H], hg[:, H:2 * H], hg[:, 2 * H:]
        r = jax.nn.sigmoid(xr + hr)
        z = jax.nn.sigmoid(xz + hz)
        nn_ = jnp.tanh(xn + r * hn)
        h_new = (1.0 - z) * nn_ + z * h
        states_ref[:, pl.ds(t, 1), :] = h_new.reshape(B, 1, H)
        return h_new

    lax.fori_loop(0, S, _step, jnp.zeros((B, H), jnp.float32))

    br = bias3[0, 0]
    bw = bias3[0, 1]
    bd = bias3[0, 2]
    ms = mscale[0, 0]
    Whe = W_he[...]
    bhe = b_he[...]
    wrd_v = wrd[...]

    iota_t = lax.broadcasted_iota(jnp.int32, (S, S), 0)
    iota_j = lax.broadcasted_iota(jnp.int32, (S, S), 1)
    mask_lt = iota_j < iota_t
    lt_le_col = jnp.where(iota_j < iota_t, 1.0, 0.0)
    le_row = jnp.where(iota_t <= iota_j, 1.0, 0.0)

    for b in range(B):
        st_b = states_ref[b, :, :]
        proj_ref[b * S:(b + 1) * S, :] = lax.dot_general(
            st_b, Whe, (((1,), (1,)), ((), ())),
            preferred_element_type=jnp.float32) + bhe
        g_col = lax.dot_general(st_b, wrd_v, (((1,), (1,)), ((), ())),
                                preferred_element_type=jnp.float32)
        g_row = lax.dot_general(wrd_v, st_b, (((1,), (1,)), ((), ())),
                                preferred_element_type=jnp.float32)
        read_col = jax.nn.sigmoid(g_col[:, 0:1] + br) * ms
        write_row = jax.nn.sigmoid(g_row[1:2, :] + bw)
        dec_col = jax.nn.sigmoid(g_col[:, 2:3] + bd)
        dec_row = jax.nn.sigmoid(g_row[2:3, :] + bd)
        ld_col = jnp.log(jnp.maximum(dec_col, 1e-30))
        ld_row = jnp.log(jnp.maximum(dec_row, 1e-30))
        cum_row = lax.dot_general(ld_row, le_row, (((1,), (0,)), ((), ())),
                                  preferred_element_type=jnp.float32)
        cum_tm1_col = lax.dot_general(lt_le_col, ld_col,
                                      (((1,), (0,)), ((), ())),
                                      preferred_element_type=jnp.float32)
        arg = jnp.where(mask_lt, cum_tm1_col - cum_row, -1e30)
        a3_ref[b, :, :] = jnp.exp(arg) * write_row * read_col

    del ids_vmem, ids_col


def _logits_body(proj, emb_blk, bias_blk, a3, ids_col, out_ref):
    base = lax.dot_general(
        proj[...], emb_blk[...], (((1,), (1,)), ((), ())),
        preferred_element_type=jnp.float32) + bias_blk[...]
    v0 = pl.program_id(0) * VBLK
    col = lax.broadcasted_iota(jnp.int32, (S, VBLK), 1) + v0
    for b in range(B):
        idb = ids_col[b, :, :]
        oh = jnp.where(col == idb, 1.0, 0.0)
        corr = lax.dot_general(a3[b, :, :], oh, (((1,), (0,)), ((), ())),
                               preferred_element_type=jnp.float32)
        out_ref[b * S:(b + 1) * S, :] = base[b * S:(b + 1) * S, :] + corr


def kernel(input_ids, emb, W_ih, W_hh, b_ih, b_hh, W_he, b_he, out_bias,
           w_read, b_read, w_write, b_write, w_decay, b_decay, mem_scale):
    ids = input_ids.astype(jnp.int32)
    ids_col = ids.reshape(B, S, 1)
    wrd = jnp.concatenate([w_read, w_write, w_decay], axis=0)
    bias3 = jnp.stack([b_read[0], b_write[0], b_decay[0]]).reshape(1, 3)
    mscale = mem_scale.reshape(1, 1)

    proj, a3 = pl.pallas_call(
        _coeffs_body,
        in_specs=[
            pl.BlockSpec(memory_space=pltpu.SMEM),
            pl.BlockSpec(memory_space=pltpu.HBM),
            pl.BlockSpec(memory_space=pltpu.VMEM),
            pl.BlockSpec(memory_space=pltpu.VMEM),
            pl.BlockSpec(memory_space=pltpu.VMEM),
            pl.BlockSpec(memory_space=pltpu.VMEM),
            pl.BlockSpec(memory_space=pltpu.VMEM),
            pl.BlockSpec(memory_space=pltpu.VMEM),
            pl.BlockSpec(memory_space=pltpu.VMEM),
            pl.BlockSpec(memory_space=pltpu.VMEM),
            pl.BlockSpec(memory_space=pltpu.VMEM),
            pl.BlockSpec(memory_space=pltpu.SMEM),
            pl.BlockSpec(memory_space=pltpu.SMEM),
        ],
        out_shape=[
            jax.ShapeDtypeStruct((N, D), jnp.float32),
            jax.ShapeDtypeStruct((B, S, S), jnp.float32),
        ],
        scratch_shapes=[
            pltpu.VMEM((N, D), jnp.float32),
            pltpu.VMEM((N, 3 * H), jnp.float32),
            pltpu.VMEM((B, S, H), jnp.float32),
            pltpu.SemaphoreType.DMA,
        ],
    )(ids, emb, ids, ids_col, W_ih, W_hh, b_ih.reshape(1, 3 * H),
      b_hh.reshape(1, 3 * H), W_he, b_he.reshape(1, D), wrd, bias3, mscale)

    nblk = (V + VBLK - 1) // VBLK
    out2d = pl.pallas_call(
        _logits_body,
        grid=(nblk,),
        in_specs=[
            pl.BlockSpec((N, D), lambda i: (0, 0)),
            pl.BlockSpec((VBLK, D), lambda i: (i, 0)),
            pl.BlockSpec((1, VBLK), lambda i: (0, i)),
            pl.BlockSpec((B, S, S), lambda i: (0, 0, 0)),
            pl.BlockSpec((B, S, 1), lambda i: (0, 0, 0)),
        ],
        out_specs=pl.BlockSpec((N, VBLK), lambda i: (0, i)),
        out_shape=jax.ShapeDtypeStruct((N, V), jnp.float32),
    )(proj, emb, out_bias.reshape(1, V), a3, ids_col)

    return out2d.reshape(B, S, V)
